# baseline (speedup 1.0000x reference)
import functools

import jax
import jax.numpy as jnp
from jax import lax
from jax.experimental import pallas as pl
from jax.experimental.pallas import tpu as pltpu
from jax.experimental.pallas import tpu_sc as plsc

_N = 50000
_E = 1600000
_D = 128
_H = 128
_C = 48
_ALPHA = 0.1
_K = 10
_A1, _A2, _A3 = 0.4, 0.3, 0.3

_N2 = _N // 2
_RPT = 1568
_ACC_ROWS = 16 * _RPT
_NP = 2 * _ACC_ROWS
_DUMP = _N2
_JCH = 3
_CHUNK_E = _JCH * 128
_CAPCH = 4192
_CAPA = _CAPCH + 1
_CAP_E = _CAPCH * _CHUNK_E
_PCH = 14
_PR = _RPT // _PCH


def _step_body(g_hbm, a_hbm, b_hbm, idx_hbm, kch_hbm, out_hbm,
               acc, kcv, idxv0, rows0, idxv1, rows1, mv, av, bv,
               gsem0, gsem1, ssem0, ssem1):
    c = lax.axis_index("c")
    s = lax.axis_index("s")
    zero16 = jnp.zeros((16,), jnp.float32)
    slots = ((idxv0, rows0, gsem0, ssem0),
             (idxv1, rows1, gsem1, ssem1))
    pltpu.sync_copy(kch_hbm.at[c], kcv)
    kch = jnp.max(kcv[...])
    base = s * (2 * kch)

    def zrow(r, _):
        for cg in range(3):
            mv[r, pl.ds(cg * 16, 16)] = zero16
        return 0
    lax.fori_loop(0, _PR, zrow, 0)
    for cc in range(_PCH):
        pltpu.sync_copy(mv, acc.at[pl.ds(s * _RPT + cc * _PR, _PR)])
    plsc.subcore_barrier()

    def load_idx(k, idxv):
        pltpu.sync_copy(idx_hbm.at[c, base + k], idxv)

    def fire_g(idxv, rows, gsem):
        for j in range(_JCH):
            pltpu.async_copy(g_hbm.at[idxv.at[0, j]],
                             rows.at[pl.ds(j * 128, 128)], gsem)

    def drain(rows, sem):
        pltpu.make_async_copy(g_hbm.at[pl.ds(0, _CHUNK_E)], rows, sem).wait()

    def fire_s(idxv, rows, ssem):
        for j in range(_JCH):
            pltpu.async_copy(rows.at[pl.ds(j * 128, 128)],
                             acc.at[idxv.at[1, j]], ssem, add=True)

    load_idx(0, idxv0)
    fire_g(idxv0, rows0, gsem0)

    def pair(kk, _):
        for par in range(2):
            k = 2 * kk + par
            idxv, rows, gsem, ssem = slots[par]
            qidxv, qrows, qgsem, qssem = slots[1 - par]
            if par == 0:
                @pl.when(kk > 0)
                def _():
                    drain(qrows, qssem)
            else:
                drain(qrows, qssem)
            load_idx(k + 1, qidxv)
            fire_g(qidxv, qrows, qgsem)
            drain(rows, gsem)
            fire_s(idxv, rows, ssem)
        return 0
    lax.fori_loop(0, kch, pair, 0)
    drain(rows1, ssem1)
    drain(rows0, gsem0)
    plsc.subcore_barrier()

    for cc in range(_PCH):
        r0 = s * _RPT + cc * _PR
        g0 = c * _ACC_ROWS + r0
        pltpu.sync_copy(acc.at[pl.ds(r0, _PR)], mv)
        pltpu.sync_copy(a_hbm.at[pl.ds(g0, _PR)], av)
        pltpu.sync_copy(b_hbm.at[pl.ds(g0, _PR)], bv)

        def prow(r, _):
            for cg in range(3):
                sl = pl.ds(cg * 16, 16)
                mv[r, sl] = mv[r, sl] * av[r, sl] + bv[r, sl]
            return 0
        lax.fori_loop(0, _PR, prow, 0)
        pltpu.sync_copy(mv, out_hbm.at[pl.ds(g0, _PR)])


@jax.jit
def _step(g_tbl, a_tbl, b_tbl, idx_arr, kch_arr):
    mesh = plsc.VectorSubcoreMesh(core_axis_name="c", subcore_axis_name="s")
    return pl.kernel(
        _step_body,
        out_type=jax.ShapeDtypeStruct((_NP, _C), jnp.float32),
        mesh=mesh,
        compiler_params=pltpu.CompilerParams(use_tc_tiling_on_sc=False, needs_layout_passes=False),
        scratch_types=[
            pltpu.VMEM_SHARED((_ACC_ROWS, _C), jnp.float32),
            pltpu.VMEM((16,), jnp.int32),
            pltpu.VMEM((2, _JCH, 128), jnp.int32),
            pltpu.VMEM((_CHUNK_E, _C), jnp.float32),
            pltpu.VMEM((2, _JCH, 128), jnp.int32),
            pltpu.VMEM((_CHUNK_E, _C), jnp.float32),
            pltpu.VMEM((_PR, _C), jnp.float32),
            pltpu.VMEM((_PR, _C), jnp.float32),
            pltpu.VMEM((_PR, _C), jnp.float32),
            pltpu.SemaphoreType.DMA,
            pltpu.SemaphoreType.DMA,
            pltpu.SemaphoreType.DMA,
            pltpu.SemaphoreType.DMA,
        ],
    )(g_tbl, a_tbl, b_tbl, idx_arr, kch_arr)


def _deg_body(idx_hbm, kch_hbm, out_hbm, acc, ones_rows, kcv, idxv0, idxv1,
              mv, ssem0, ssem1):
    c = lax.axis_index("c")
    s = lax.axis_index("s")
    zero16 = jnp.zeros((16,), jnp.float32)
    one16 = jnp.ones((16,), jnp.float32)
    pltpu.sync_copy(kch_hbm.at[c], kcv)
    kch = jnp.max(kcv[...])
    base = s * (2 * kch)

    def zrow(r, _):
        for cg in range(3):
            mv[r, pl.ds(cg * 16, 16)] = zero16
        return 0
    lax.fori_loop(0, _PR, zrow, 0)
    for cc in range(_PCH):
        pltpu.sync_copy(mv, acc.at[pl.ds(s * _RPT + cc * _PR, _PR)])

    def orow(r, _):
        for cg in range(3):
            ones_rows[r, pl.ds(cg * 16, 16)] = one16
        return 0
    lax.fori_loop(0, 128, orow, 0)
    plsc.subcore_barrier()

    def fire_s(idxv, ssem):
        for j in range(_JCH):
            pltpu.async_copy(ones_rows, acc.at[idxv.at[1, j]], ssem, add=True)

    def wait_s(ssem):
        for j in range(_JCH):
            pltpu.make_async_copy(out_hbm.at[pl.ds(0, 128)], ones_rows,
                                  ssem).wait()

    slots = ((idxv0, ssem0), (idxv1, ssem1))
    pltpu.sync_copy(idx_hbm.at[c, base], idxv0)

    def pair(kk, _):
        for par in range(2):
            k = 2 * kk + par
            idxv, ssem = slots[par]
            qidxv, qssem = slots[1 - par]
            if par == 0:
                @pl.when(kk > 0)
                def _():
                    wait_s(qssem)
            else:
                wait_s(qssem)
            pltpu.sync_copy(idx_hbm.at[c, base + k + 1], qidxv)
            fire_s(idxv, ssem)
        return 0
    lax.fori_loop(0, kch, pair, 0)
    wait_s(ssem1)
    plsc.subcore_barrier()

    for cc in range(_PCH):
        r0 = s * _RPT + cc * _PR
        g0 = c * _ACC_ROWS + r0
        pltpu.sync_copy(acc.at[pl.ds(r0, _PR)], mv)
        pltpu.sync_copy(mv, out_hbm.at[pl.ds(g0, _PR)])


@jax.jit
def _deg(idx_arr, kch_arr):
    mesh = plsc.VectorSubcoreMesh(core_axis_name="c", subcore_axis_name="s")
    return pl.kernel(
        _deg_body,
        out_type=jax.ShapeDtypeStruct((_NP, _C), jnp.float32),
        mesh=mesh,
        compiler_params=pltpu.CompilerParams(use_tc_tiling_on_sc=False, needs_layout_passes=False),
        scratch_types=[
            pltpu.VMEM_SHARED((_ACC_ROWS, _C), jnp.float32),
            pltpu.VMEM((128, _C), jnp.float32),
            pltpu.VMEM((16,), jnp.int32),
            pltpu.VMEM((2, _JCH, 128), jnp.int32),
            pltpu.VMEM((2, _JCH, 128), jnp.int32),
            pltpu.VMEM((_PR, _C), jnp.float32),
            pltpu.SemaphoreType.DMA,
            pltpu.SemaphoreType.DMA,
        ],
    )(idx_arr, kch_arr)


def _mlp_body(f1, f2, f3, deg,
              w1a, b1a, w1b, b1b, w2a, b2a, w2b, b2b, w3a, b3a, w3b, b3b,
              g0_o, nsq_o, ah0n_o, fina_o, finb_o):
    h = jnp.zeros_like(g0_o)
    for x_ref, wa, ba, wb, bb, aw in (
            (f1, w1a, b1a, w1b, b1b, _A1),
            (f2, w2a, b2a, w2b, b2b, _A2),
            (f3, w3a, b3a, w3b, b3b, _A3)):
        t = jnp.maximum(
            jnp.dot(x_ref[...], wa[...], preferred_element_type=jnp.float32)
            + ba[...], 0.0)
        h = h + aw * (jnp.dot(t, wb[...], preferred_element_type=jnp.float32)
                      + bb[...])
    norm = lax.rsqrt(jnp.clip(deg[...], 1.0, None))
    one_m_a = 1.0 - _ALPHA
    g0_o[...] = h * norm
    nsq_o[...] = one_m_a * norm * norm
    ah0n_o[...] = _ALPHA * h * norm
    fina_o[...] = one_m_a * norm
    finb_o[...] = _ALPHA * h


@jax.jit
def _mlp(f1p, f2p, f3p, deg48,
         w1a, b1a, w1b, b1b, w2a, b2a, w2b, b2b, w3a, b3a, w3b, b3b):
    blk = 512
    grid = (_NP // blk,)
    fspec = pl.BlockSpec((blk, _D), lambda i: (i, 0))
    dspec = pl.BlockSpec((blk, _C), lambda i: (i, 0))
    waspec = pl.BlockSpec((_D, _H), lambda i: (0, 0))
    baspec = pl.BlockSpec((1, _H), lambda i: (0, 0))
    wbspec = pl.BlockSpec((_H, _C), lambda i: (0, 0))
    bbspec = pl.BlockSpec((1, _C), lambda i: (0, 0))
    ospec = pl.BlockSpec((blk, _C), lambda i: (i, 0))
    out = jax.ShapeDtypeStruct((_NP, _C), jnp.float32)
    return pl.pallas_call(
        _mlp_body,
        grid=grid,
        in_specs=[fspec, fspec, fspec, dspec] + [waspec, baspec, wbspec, bbspec] * 3,
        out_specs=[ospec] * 5,
        out_shape=[out] * 5,
    )(f1p, f2p, f3p, deg48,
      w1a, b1a.reshape(1, _H), w1b, b1b.reshape(1, _C),
      w2a, b2a.reshape(1, _H), w2b, b2b.reshape(1, _C),
      w3a, b3a.reshape(1, _H), w3b, b3b.reshape(1, _C))


def kernel(features1, features2, features3, edge_index,
           W1a, b1a, W1b, b1b, W2a, b2a, W2b, b2b, W3a, b3a, W3b, b3b):
    src = edge_index[0].astype(jnp.int32)
    dst = edge_index[1].astype(jnp.int32)

    srcp = src + jnp.where(src >= _N2, _ACC_ROWS - _N2, 0).astype(jnp.int32)

    key = (dst >= _N2).astype(jnp.int32)
    perm = jnp.argsort(key).astype(jnp.int32)
    srcp_s = jnp.take(srcp, perm)
    dstl_s = jnp.take(dst - key * _N2, perm)
    cnt1 = jnp.sum(key, dtype=jnp.int32)
    cnt0 = jnp.int32(_E) - cnt1

    slot = jnp.arange(_CAP_E, dtype=jnp.int32)

    def pack(start, cnt):
        pos = jnp.clip(start + slot, 0, _E - 1)
        valid = slot < cnt
        s_ = jnp.where(valid, jnp.take(srcp_s, pos), 0)
        d_ = jnp.where(valid, jnp.take(dstl_s, pos), _DUMP)
        blk = jnp.stack([s_.reshape(_CAPCH, _JCH, 128),
                         d_.reshape(_CAPCH, _JCH, 128)], axis=1)
        dummy = jnp.zeros((1, 2, _JCH, 128), jnp.int32)
        return jnp.concatenate([blk, dummy], axis=0)

    idx_arr = jnp.stack([pack(jnp.int32(0), cnt0), pack(cnt0, cnt1)])

    def pairs(cnt):
        return jnp.maximum(1, (cnt + 2 * 16 * _CHUNK_E - 1)
                           // (2 * 16 * _CHUNK_E)).astype(jnp.int32)

    kch_arr = jnp.broadcast_to(
        jnp.stack([pairs(cnt0), pairs(cnt1)])[:, None], (2, 16))

    deg48 = _deg(idx_arr, kch_arr)

    z88 = jnp.zeros((_ACC_ROWS - _N2, _D), jnp.float32)

    def remap(f):
        return jnp.concatenate([f[:_N2], z88, f[_N2:], z88], axis=0)

    g, nsq, ah0n, fina, finb = _mlp(
        remap(features1), remap(features2), remap(features3), deg48,
        W1a, b1a, W1b, b1b, W2a, b2a, W2b, b2b, W3a, b3a, W3b, b3b)

    for _ in range(_K - 1):
        g = _step(g, nsq, ah0n, idx_arr, kch_arr)
    hp = _step(g, fina, finb, idx_arr, kch_arr)
    return jnp.concatenate([hp[:_N2], hp[_ACC_ROWS:_ACC_ROWS + _N2]], axis=0)


# speedup vs baseline: 1.9849x; 1.9849x over previous
import functools

import jax
import jax.numpy as jnp
from jax import lax
from jax.experimental import pallas as pl
from jax.experimental.pallas import tpu as pltpu
from jax.experimental.pallas import tpu_sc as plsc

_N = 50000
_E = 1600000
_D = 128
_H = 128
_C = 48
_ALPHA = 0.1
_K = 10
_A1, _A2, _A3 = 0.4, 0.3, 0.3

_N2 = _N // 2
_RPT = 1568
_ACC_ROWS = 16 * _RPT
_NP = 2 * _ACC_ROWS
_DUMP = _N2
_JCH = 3
_CHUNK_E = _JCH * 128
_CAPCH = 4192
_CAPA = _CAPCH + 1
_CAP_E = _CAPCH * _CHUNK_E
_PCH = 14
_PR = _RPT // _PCH


def _step_body(g_hbm, a_hbm, b_hbm, idx_hbm, kch_hbm, out_hbm,
               acc, kcv, cv0, srcv0, dstv0, rows0, cv1, srcv1, dstv1, rows1,
               mv, av, bv, gsem0, gsem1, ssem0, ssem1):
    c = lax.axis_index("c")
    s = lax.axis_index("s")
    zero16 = jnp.zeros((16,), jnp.float32)
    slots = ((cv0, srcv0, dstv0, rows0, gsem0, ssem0),
             (cv1, srcv1, dstv1, rows1, gsem1, ssem1))
    pltpu.sync_copy(kch_hbm.at[c], kcv)
    kch = jnp.max(kcv[...])
    base = s * (2 * kch)
    dsub = c * _N2

    def zrow(r, _):
        for cg in range(3):
            mv[r, pl.ds(cg * 16, 16)] = zero16
        return 0
    lax.fori_loop(0, _PR, zrow, 0)
    for cc in range(_PCH):
        pltpu.sync_copy(mv, acc.at[pl.ds(s * _RPT + cc * _PR, _PR)])
    plsc.subcore_barrier()

    def load_idx(k, cv, srcv, dstv):
        pltpu.sync_copy(idx_hbm.at[c, base + k], cv)
        for j in range(_JCH):
            for t in range(8):
                sl = pl.ds(t * 16, 16)
                v = cv[j, sl]
                srcv[j, sl] = (v & jnp.uint32(0xFFFF)).astype(jnp.int32)
                dstv[j, sl] = (v >> 16).astype(jnp.int32) - dsub

    def fire_g(srcv, rows, gsem):
        for j in range(_JCH):
            pltpu.async_copy(g_hbm.at[srcv.at[j]],
                             rows.at[pl.ds(j * 128, 128)], gsem)

    def drain(rows, sem):
        pltpu.make_async_copy(g_hbm.at[pl.ds(0, _CHUNK_E)], rows, sem).wait()

    def fire_s(dstv, rows, ssem):
        for j in range(_JCH):
            pltpu.async_copy(rows.at[pl.ds(j * 128, 128)],
                             acc.at[dstv.at[j]], ssem, add=True)

    load_idx(0, cv0, srcv0, dstv0)
    fire_g(srcv0, rows0, gsem0)

    def pair(kk, _):
        for par in range(2):
            k = 2 * kk + par
            cv, srcv, dstv, rows, gsem, ssem = slots[par]
            qcv, qsrcv, qdstv, qrows, qgsem, qssem = slots[1 - par]
            if par == 0:
                @pl.when(kk > 0)
                def _():
                    drain(qrows, qssem)
            else:
                drain(qrows, qssem)
            load_idx(k + 1, qcv, qsrcv, qdstv)
            fire_g(qsrcv, qrows, qgsem)
            drain(rows, gsem)
            fire_s(dstv, rows, ssem)
        return 0
    lax.fori_loop(0, kch, pair, 0)
    drain(rows1, ssem1)
    drain(rows0, gsem0)
    plsc.subcore_barrier()

    for cc in range(_PCH):
        r0 = s * _RPT + cc * _PR
        g0 = c * _ACC_ROWS + r0
        pltpu.sync_copy(acc.at[pl.ds(r0, _PR)], mv)
        pltpu.sync_copy(a_hbm.at[pl.ds(g0, _PR)], av)
        pltpu.sync_copy(b_hbm.at[pl.ds(g0, _PR)], bv)

        def prow(r, _):
            for cg in range(3):
                sl = pl.ds(cg * 16, 16)
                mv[r, sl] = mv[r, sl] * av[r, sl] + bv[r, sl]
            return 0
        lax.fori_loop(0, _PR, prow, 0)
        pltpu.sync_copy(mv, out_hbm.at[pl.ds(g0, _PR)])


@jax.jit
def _step(g_tbl, a_tbl, b_tbl, idx_arr, kch_arr):
    mesh = plsc.VectorSubcoreMesh(core_axis_name="c", subcore_axis_name="s")
    return pl.kernel(
        _step_body,
        out_type=jax.ShapeDtypeStruct((_NP, _C), jnp.float32),
        mesh=mesh,
        compiler_params=pltpu.CompilerParams(use_tc_tiling_on_sc=False, needs_layout_passes=False),
        scratch_types=[
            pltpu.VMEM_SHARED((_ACC_ROWS, _C), jnp.float32),
            pltpu.VMEM((16,), jnp.int32),
            pltpu.VMEM((_JCH, 128), jnp.uint32),
            pltpu.VMEM((_JCH, 128), jnp.int32),
            pltpu.VMEM((_JCH, 128), jnp.int32),
            pltpu.VMEM((_CHUNK_E, _C), jnp.float32),
            pltpu.VMEM((_JCH, 128), jnp.uint32),
            pltpu.VMEM((_JCH, 128), jnp.int32),
            pltpu.VMEM((_JCH, 128), jnp.int32),
            pltpu.VMEM((_CHUNK_E, _C), jnp.float32),
            pltpu.VMEM((_PR, _C), jnp.float32),
            pltpu.VMEM((_PR, _C), jnp.float32),
            pltpu.VMEM((_PR, _C), jnp.float32),
            pltpu.SemaphoreType.DMA,
            pltpu.SemaphoreType.DMA,
            pltpu.SemaphoreType.DMA,
            pltpu.SemaphoreType.DMA,
        ],
    )(g_tbl, a_tbl, b_tbl, idx_arr, kch_arr)


def _deg_body(idx_hbm, kch_hbm, out_hbm, acc, ones_rows, kcv, cv0, dstv0,
              cv1, dstv1, mv, ssem0, ssem1):
    c = lax.axis_index("c")
    s = lax.axis_index("s")
    zero16 = jnp.zeros((16,), jnp.float32)
    one16 = jnp.ones((16,), jnp.float32)
    pltpu.sync_copy(kch_hbm.at[c], kcv)
    kch = jnp.max(kcv[...])
    base = s * (2 * kch)
    dsub = c * _N2

    def zrow(r, _):
        for cg in range(3):
            mv[r, pl.ds(cg * 16, 16)] = zero16
        return 0
    lax.fori_loop(0, _PR, zrow, 0)
    for cc in range(_PCH):
        pltpu.sync_copy(mv, acc.at[pl.ds(s * _RPT + cc * _PR, _PR)])

    def orow(r, _):
        for cg in range(3):
            ones_rows[r, pl.ds(cg * 16, 16)] = one16
        return 0
    lax.fori_loop(0, 128, orow, 0)
    plsc.subcore_barrier()

    def load_idx(k, cv, dstv):
        pltpu.sync_copy(idx_hbm.at[c, base + k], cv)
        for j in range(_JCH):
            for t in range(8):
                sl = pl.ds(t * 16, 16)
                dstv[j, sl] = (cv[j, sl] >> 16).astype(jnp.int32) - dsub

    def fire_s(dstv, ssem):
        for j in range(_JCH):
            pltpu.async_copy(ones_rows, acc.at[dstv.at[j]], ssem, add=True)

    def wait_s(ssem):
        for j in range(_JCH):
            pltpu.make_async_copy(out_hbm.at[pl.ds(0, 128)], ones_rows,
                                  ssem).wait()

    slots = ((cv0, dstv0, ssem0), (cv1, dstv1, ssem1))
    load_idx(0, cv0, dstv0)

    def pair(kk, _):
        for par in range(2):
            k = 2 * kk + par
            cv, dstv, ssem = slots[par]
            qcv, qdstv, qssem = slots[1 - par]
            if par == 0:
                @pl.when(kk > 0)
                def _():
                    wait_s(qssem)
            else:
                wait_s(qssem)
            load_idx(k + 1, qcv, qdstv)
            fire_s(dstv, ssem)
        return 0
    lax.fori_loop(0, kch, pair, 0)
    wait_s(ssem1)
    plsc.subcore_barrier()

    for cc in range(_PCH):
        r0 = s * _RPT + cc * _PR
        g0 = c * _ACC_ROWS + r0
        pltpu.sync_copy(acc.at[pl.ds(r0, _PR)], mv)
        pltpu.sync_copy(mv, out_hbm.at[pl.ds(g0, _PR)])


@jax.jit
def _deg(idx_arr, kch_arr):
    mesh = plsc.VectorSubcoreMesh(core_axis_name="c", subcore_axis_name="s")
    return pl.kernel(
        _deg_body,
        out_type=jax.ShapeDtypeStruct((_NP, _C), jnp.float32),
        mesh=mesh,
        compiler_params=pltpu.CompilerParams(use_tc_tiling_on_sc=False, needs_layout_passes=False),
        scratch_types=[
            pltpu.VMEM_SHARED((_ACC_ROWS, _C), jnp.float32),
            pltpu.VMEM((128, _C), jnp.float32),
            pltpu.VMEM((16,), jnp.int32),
            pltpu.VMEM((_JCH, 128), jnp.uint32),
            pltpu.VMEM((_JCH, 128), jnp.int32),
            pltpu.VMEM((_JCH, 128), jnp.uint32),
            pltpu.VMEM((_JCH, 128), jnp.int32),
            pltpu.VMEM((_PR, _C), jnp.float32),
            pltpu.SemaphoreType.DMA,
            pltpu.SemaphoreType.DMA,
        ],
    )(idx_arr, kch_arr)


def _mlp_body(f1, f2, f3, deg,
              w1a, b1a, w1b, b1b, w2a, b2a, w2b, b2b, w3a, b3a, w3b, b3b,
              g0_o, nsq_o, ah0n_o, fina_o, finb_o):
    h = jnp.zeros_like(g0_o)
    for x_ref, wa, ba, wb, bb, aw in (
            (f1, w1a, b1a, w1b, b1b, _A1),
            (f2, w2a, b2a, w2b, b2b, _A2),
            (f3, w3a, b3a, w3b, b3b, _A3)):
        t = jnp.maximum(
            jnp.dot(x_ref[...], wa[...], preferred_element_type=jnp.float32)
            + ba[...], 0.0)
        h = h + aw * (jnp.dot(t, wb[...], preferred_element_type=jnp.float32)
                      + bb[...])
    norm = lax.rsqrt(jnp.clip(deg[...], 1.0, None))
    one_m_a = 1.0 - _ALPHA
    g0_o[...] = h * norm
    nsq_o[...] = one_m_a * norm * norm
    ah0n_o[...] = _ALPHA * h * norm
    fina_o[...] = one_m_a * norm
    finb_o[...] = _ALPHA * h


@jax.jit
def _mlp(f1p, f2p, f3p, deg48,
         w1a, b1a, w1b, b1b, w2a, b2a, w2b, b2b, w3a, b3a, w3b, b3b):
    blk = 512
    grid = (_NP // blk,)
    fspec = pl.BlockSpec((blk, _D), lambda i: (i, 0))
    dspec = pl.BlockSpec((blk, _C), lambda i: (i, 0))
    waspec = pl.BlockSpec((_D, _H), lambda i: (0, 0))
    baspec = pl.BlockSpec((1, _H), lambda i: (0, 0))
    wbspec = pl.BlockSpec((_H, _C), lambda i: (0, 0))
    bbspec = pl.BlockSpec((1, _C), lambda i: (0, 0))
    ospec = pl.BlockSpec((blk, _C), lambda i: (i, 0))
    out = jax.ShapeDtypeStruct((_NP, _C), jnp.float32)
    return pl.pallas_call(
        _mlp_body,
        grid=grid,
        in_specs=[fspec, fspec, fspec, dspec] + [waspec, baspec, wbspec, bbspec] * 3,
        out_specs=[ospec] * 5,
        out_shape=[out] * 5,
    )(f1p, f2p, f3p, deg48,
      w1a, b1a.reshape(1, _H), w1b, b1b.reshape(1, _C),
      w2a, b2a.reshape(1, _H), w2b, b2b.reshape(1, _C),
      w3a, b3a.reshape(1, _H), w3b, b3b.reshape(1, _C))


def kernel(features1, features2, features3, edge_index,
           W1a, b1a, W1b, b1b, W2a, b2a, W2b, b2b, W3a, b3a, W3b, b3b):
    src = edge_index[0].astype(jnp.int32)
    dst = edge_index[1].astype(jnp.int32)

    srcp = src + jnp.where(src >= _N2, _ACC_ROWS - _N2, 0).astype(jnp.int32)

    comb = ((dst.astype(jnp.uint32) << 16)
            | srcp.astype(jnp.uint32))
    comb_s = jnp.sort(comb)
    cnt1 = jnp.sum((dst >= _N2).astype(jnp.int32), dtype=jnp.int32)
    cnt0 = jnp.int32(_E) - cnt1

    slot = jnp.arange(_CAP_E, dtype=jnp.int32)
    padded = jnp.concatenate(
        [comb_s, jnp.zeros((_CAP_E,), jnp.uint32)])

    def pack(start, cnt, fill):
        sl = lax.dynamic_slice(padded, (start,), (_CAP_E,))
        v = jnp.where(slot < cnt, sl, fill)
        blk = v.reshape(_CAPCH, _JCH, 128)
        dummy = jnp.full((1, _JCH, 128), fill, jnp.uint32)
        return jnp.concatenate([blk, dummy], axis=0)

    fill0 = jnp.uint32(_DUMP << 16)
    fill1 = jnp.uint32((_N2 + _DUMP) << 16)
    idx_arr = jnp.stack([pack(jnp.int32(0), cnt0, fill0),
                         pack(cnt0, cnt1, fill1)])

    def pairs(cnt):
        return jnp.maximum(1, (cnt + 2 * 16 * _CHUNK_E - 1)
                           // (2 * 16 * _CHUNK_E)).astype(jnp.int32)

    kch_arr = jnp.broadcast_to(
        jnp.stack([pairs(cnt0), pairs(cnt1)])[:, None], (2, 16))

    deg48 = _deg(idx_arr, kch_arr)

    z88 = jnp.zeros((_ACC_ROWS - _N2, _D), jnp.float32)

    def remap(f):
        return jnp.concatenate([f[:_N2], z88, f[_N2:], z88], axis=0)

    g, nsq, ah0n, fina, finb = _mlp(
        remap(features1), remap(features2), remap(features3), deg48,
        W1a, b1a, W1b, b1b, W2a, b2a, W2b, b2b, W3a, b3a, W3b, b3b)

    for _ in range(_K - 1):
        g = _step(g, nsq, ah0n, idx_arr, kch_arr)
    hp = _step(g, fina, finb, idx_arr, kch_arr)
    return jnp.concatenate([hp[:_N2], hp[_ACC_ROWS:_ACC_ROWS + _N2]], axis=0)
